# baseline (speedup 1.0000x reference)
import functools

import jax
import jax.numpy as jnp
from jax.experimental import pallas as pl

HIDDEN = 1024
FFN = 3584
E = 8
TOP_K = 2
NEG_INF = -1e30

F_T = 1792
NF = FFN // F_T


def _router_kernel(x_ref, gw_ref, logits_ref, c_ref):
    x = x_ref[...]
    gw = gw_ref[...]
    logits = jax.lax.dot_general(
        x, gw, (((1,), (1,)), ((), ())),
        preferred_element_type=jnp.float32)
    logits_ref[...] = logits
    m1 = jnp.max(logits, axis=1, keepdims=True)
    l2 = jnp.where(logits == m1, NEG_INF, logits)
    m2 = jnp.max(l2, axis=1, keepdims=True)
    e2 = jnp.exp(m2 - m1)
    c = jnp.exp(logits - m1) / (1.0 + e2)
    c_ref[...] = jnp.where(logits >= m2, c, 0.0)


def _moe_kernel(x_ref, w1_ref, w3_ref, w2_ref, c_ref, out_ref):
    e = pl.program_id(0)
    f = pl.program_id(1)
    x = x_ref[...]
    w1 = w1_ref[0]
    w3 = w3_ref[0]
    w2 = w2_ref[0]
    h1 = jax.lax.dot_general(x, w1, (((1,), (1,)), ((), ())),
                             preferred_element_type=jnp.float32)
    h1 = h1 * jax.nn.sigmoid(h1)
    h3 = jax.lax.dot_general(x, w3, (((1,), (1,)), ((), ())),
                             preferred_element_type=jnp.float32)
    h = h1 * h3
    o = jax.lax.dot_general(h, w2, (((1,), (1,)), ((), ())),
                            preferred_element_type=jnp.float32)
    c = c_ref[...]
    cols = jax.lax.broadcasted_iota(jnp.int32, c.shape, 1)
    ce = jnp.sum(jnp.where(cols == e, c, 0.0), axis=1, keepdims=True)
    contrib = o * ce

    @pl.when(jnp.logical_and(e == 0, f == 0))
    def _init():
        out_ref[...] = contrib

    @pl.when(jnp.logical_or(e != 0, f != 0))
    def _acc():
        out_ref[...] += contrib


def kernel(hidden_states, gate_w, w1, w2, w3):
    batch, seq, hidden = hidden_states.shape
    x = hidden_states.reshape(-1, hidden)
    T = x.shape[0]

    logits, c = pl.pallas_call(
        _router_kernel,
        out_shape=(
            jax.ShapeDtypeStruct((T, E), jnp.float32),
            jax.ShapeDtypeStruct((T, E), jnp.float32),
        ),
    )(x, gate_w)

    out = pl.pallas_call(
        _moe_kernel,
        grid=(E, NF),
        in_specs=[
            pl.BlockSpec((T, HIDDEN), lambda e, f: (0, 0)),
            pl.BlockSpec((1, F_T, HIDDEN), lambda e, f: (e, f, 0)),
            pl.BlockSpec((1, F_T, HIDDEN), lambda e, f: (e, f, 0)),
            pl.BlockSpec((1, HIDDEN, F_T), lambda e, f: (e, 0, f)),
            pl.BlockSpec((T, E), lambda e, f: (0, 0)),
        ],
        out_specs=pl.BlockSpec((T, HIDDEN), lambda e, f: (0, 0)),
        out_shape=jax.ShapeDtypeStruct((T, HIDDEN), jnp.float32),
    )(x, w1, w3, w2, c)

    return out.reshape(batch, seq, hidden), logits


# speedup vs baseline: 1.0001x; 1.0001x over previous
import jax
import jax.numpy as jnp
from jax.experimental import pallas as pl
from jax.experimental.pallas import tpu as pltpu

HIDDEN = 1024
FFN = 3584
E = 8
TOP_K = 2
NEG_INF = -1e30

F_T = 1792
NF = FFN // F_T


def _moe_kernel(x_ref, gw_ref, w1_ref, w3_ref, w2_ref,
                out_ref, logits_ref, c_ref):
    e = pl.program_id(0)
    f = pl.program_id(1)
    x = x_ref[...]

    @pl.when(jnp.logical_and(e == 0, f == 0))
    def _router():
        gw = gw_ref[...]
        logits = jax.lax.dot_general(
            x, gw, (((1,), (1,)), ((), ())),
            preferred_element_type=jnp.float32)
        logits_ref[...] = logits
        m1 = jnp.max(logits, axis=1, keepdims=True)
        l2 = jnp.where(logits == m1, NEG_INF, logits)
        m2 = jnp.max(l2, axis=1, keepdims=True)
        e2 = jnp.exp(m2 - m1)
        c = jnp.exp(logits - m1) / (1.0 + e2)
        c_ref[...] = jnp.where(logits >= m2, c, 0.0)

    w1 = w1_ref[0]
    w3 = w3_ref[0]
    w2 = w2_ref[0]
    h1 = jax.lax.dot_general(x, w1, (((1,), (1,)), ((), ())),
                             preferred_element_type=jnp.float32)
    h1 = h1 * jax.nn.sigmoid(h1)
    h3 = jax.lax.dot_general(x, w3, (((1,), (1,)), ((), ())),
                             preferred_element_type=jnp.float32)
    h = h1 * h3
    o = jax.lax.dot_general(h, w2, (((1,), (1,)), ((), ())),
                            preferred_element_type=jnp.float32)
    c = c_ref[...]
    cols = jax.lax.broadcasted_iota(jnp.int32, c.shape, 1)
    ce = jnp.sum(jnp.where(cols == e, c, 0.0), axis=1, keepdims=True)
    contrib = o * ce

    @pl.when(jnp.logical_and(e == 0, f == 0))
    def _init():
        out_ref[...] = contrib

    @pl.when(jnp.logical_or(e != 0, f != 0))
    def _acc():
        out_ref[...] += contrib


def kernel(hidden_states, gate_w, w1, w2, w3):
    batch, seq, hidden = hidden_states.shape
    x = hidden_states.reshape(-1, hidden)
    T = x.shape[0]

    out, logits = pl.pallas_call(
        _moe_kernel,
        grid=(E, NF),
        in_specs=[
            pl.BlockSpec((T, HIDDEN), lambda e, f: (0, 0)),
            pl.BlockSpec((E, HIDDEN), lambda e, f: (0, 0)),
            pl.BlockSpec((1, F_T, HIDDEN), lambda e, f: (e, f, 0)),
            pl.BlockSpec((1, F_T, HIDDEN), lambda e, f: (e, f, 0)),
            pl.BlockSpec((1, HIDDEN, F_T), lambda e, f: (e, 0, f)),
        ],
        out_specs=(
            pl.BlockSpec((T, HIDDEN), lambda e, f: (0, 0)),
            pl.BlockSpec((T, E), lambda e, f: (0, 0)),
        ),
        out_shape=(
            jax.ShapeDtypeStruct((T, HIDDEN), jnp.float32),
            jax.ShapeDtypeStruct((T, E), jnp.float32),
        ),
        scratch_shapes=[pltpu.VMEM((T, E), jnp.float32)],
    )(x, gate_w, w1, w3, w2)

    return out.reshape(batch, seq, hidden), logits


# speedup vs baseline: 1.0061x; 1.0060x over previous
import jax
import jax.numpy as jnp
from jax.experimental import pallas as pl
from jax.experimental.pallas import tpu as pltpu

HIDDEN = 1024
FFN = 3584
E = 8
TOP_K = 2
NEG_INF = -1e30

F_T = 1792
NF = FFN // F_T


def _moe_kernel(x_ref, gw_ref, w1_ref, w3_ref, w2_ref,
                out_ref, logits_ref, c_ref):
    e = pl.program_id(0)
    f = pl.program_id(1)
    x = x_ref[...]

    @pl.when(jnp.logical_and(e == 0, f == 0))
    def _router():
        gw = gw_ref[...]
        logits = jax.lax.dot_general(
            x, gw, (((1,), (1,)), ((), ())),
            preferred_element_type=jnp.float32)
        logits_ref[...] = logits
        m1 = jnp.max(logits, axis=1, keepdims=True)
        l2 = jnp.where(logits == m1, NEG_INF, logits)
        m2 = jnp.max(l2, axis=1, keepdims=True)
        e2 = jnp.exp(m2 - m1)
        c = jnp.exp(logits - m1) / (1.0 + e2)
        c_ref[...] = jnp.where(logits >= m2, c, 0.0)

    xb = x.astype(jnp.bfloat16)
    w1 = w1_ref[0].astype(jnp.bfloat16)
    w3 = w3_ref[0].astype(jnp.bfloat16)
    w2 = w2_ref[0].astype(jnp.bfloat16)
    h1 = jax.lax.dot_general(xb, w1, (((1,), (1,)), ((), ())),
                             preferred_element_type=jnp.float32)
    h1 = h1 * jax.nn.sigmoid(h1)
    h3 = jax.lax.dot_general(xb, w3, (((1,), (1,)), ((), ())),
                             preferred_element_type=jnp.float32)
    h = (h1 * h3).astype(jnp.bfloat16)
    o = jax.lax.dot_general(h, w2, (((1,), (1,)), ((), ())),
                            preferred_element_type=jnp.float32)
    c = c_ref[...]
    cols = jax.lax.broadcasted_iota(jnp.int32, c.shape, 1)
    ce = jnp.sum(jnp.where(cols == e, c, 0.0), axis=1, keepdims=True)
    contrib = o * ce

    @pl.when(jnp.logical_and(e == 0, f == 0))
    def _init():
        out_ref[...] = contrib

    @pl.when(jnp.logical_or(e != 0, f != 0))
    def _acc():
        out_ref[...] += contrib


def kernel(hidden_states, gate_w, w1, w2, w3):
    batch, seq, hidden = hidden_states.shape
    x = hidden_states.reshape(-1, hidden)
    T = x.shape[0]

    out, logits = pl.pallas_call(
        _moe_kernel,
        grid=(E, NF),
        in_specs=[
            pl.BlockSpec((T, HIDDEN), lambda e, f: (0, 0)),
            pl.BlockSpec((E, HIDDEN), lambda e, f: (0, 0)),
            pl.BlockSpec((1, F_T, HIDDEN), lambda e, f: (e, f, 0)),
            pl.BlockSpec((1, F_T, HIDDEN), lambda e, f: (e, f, 0)),
            pl.BlockSpec((1, HIDDEN, F_T), lambda e, f: (e, 0, f)),
        ],
        out_specs=(
            pl.BlockSpec((T, HIDDEN), lambda e, f: (0, 0)),
            pl.BlockSpec((T, E), lambda e, f: (0, 0)),
        ),
        out_shape=(
            jax.ShapeDtypeStruct((T, HIDDEN), jnp.float32),
            jax.ShapeDtypeStruct((T, E), jnp.float32),
        ),
        scratch_shapes=[pltpu.VMEM((T, E), jnp.float32)],
    )(x, gate_w, w1, w3, w2)

    return out.reshape(batch, seq, hidden), logits
